# Initial kernel scaffold; baseline (speedup 1.0000x reference)
import jax
import jax.numpy as jnp
from jax.experimental import pallas as pl


def kernel(x_0, W, att_weight, incidence_indices):
    raise NotImplementedError("write your pallas kernel here")


# speedup vs baseline: 3.7182x; 3.7182x over previous
import functools

import jax
import jax.numpy as jnp
from jax import lax
from jax.experimental import pallas as pl
from jax.experimental.pallas import tpu as pltpu
from jax.experimental.pallas import tpu_sc as plsc

N_NODES = 10000
N_EDGES = 10000
NNZ = 320000
D = 128

NC = 2
NS = 16
NW = NC * NS
K = 96
NJ = 106
NNZ_PAD = NJ * NW * K
ACC_ROWS = 10112
ZROWS = ACC_ROWS // NS
OROWS = 624

def _common_scratch():
    return [
        pltpu.VMEM((K,), jnp.int32),
        pltpu.VMEM((K,), jnp.int32),
        pltpu.VMEM((K,), jnp.int32),
        pltpu.VMEM((K,), jnp.int32),
        pltpu.VMEM((K, D), jnp.float32),
        pltpu.VMEM((K, D), jnp.float32),
        pltpu.VMEM_SHARED((ACC_ROWS, D), jnp.float32),
        pltpu.SemaphoreType.DMA,
        pltpu.SemaphoreType.DMA,
    ]


def _pipeline(table_hbm, gidx_hbm, sidx_hbm, zeros_hbm, out_hbm,
              gi, si, rb, acc, sem, scale_fn):
    c = lax.axis_index("c")
    s = lax.axis_index("s")
    t = s * NC + c

    pltpu.sync_copy(zeros_hbm, acc.at[pl.ds(pl.multiple_of(s * ZROWS, 8),
                                            ZROWS)])
    plsc.subcore_barrier()

    def idx_copy(j, b):
        off = pl.multiple_of((j * NW + t) * K, 8)
        pltpu.sync_copy(gidx_hbm.at[pl.ds(off, K)], gi[b])
        pltpu.sync_copy(sidx_hbm.at[pl.ds(off, K)], si[b])

    def gather_start(b):
        pltpu.async_copy(table_hbm.at[gi[b]], rb[b], sem[b])

    def gather_wait(b):
        pltpu.make_async_copy(table_hbm.at[gi[b]], rb[b], sem[b]).wait()

    def scatter(b):
        pltpu.sync_copy(rb[b], acc.at[si[b]], add=True)

    idx_copy(0, 0)
    gather_start(0)
    idx_copy(1, 1)

    def pair(p, carry):
        for b in range(2):
            j = p * 2 + b
            gather_wait(b)
            gather_start(1 - b)
            scale_fn(b, gi, si, rb)
            scatter(b)
            idx_copy(j + 2, b)
        return carry

    lax.fori_loop(0, NJ // 2 - 1, pair, 0)

    gather_wait(0)
    gather_start(1)
    scale_fn(0, gi, si, rb)
    scatter(0)
    gather_wait(1)
    scale_fn(1, gi, si, rb)
    scatter(1)

    plsc.subcore_barrier()

    @pl.when(s < NS - 1)
    def _():
        pltpu.sync_copy(
            acc.at[pl.ds(pl.multiple_of(s * OROWS, 8), OROWS)],
            out_hbm.at[pl.ds(pl.multiple_of(c * N_EDGES + s * OROWS, 8),
                             OROWS)])

    last = (NS - 1) * OROWS
    last_n = N_EDGES - last

    @pl.when(s == NS - 1)
    def _():
        pltpu.sync_copy(
            acc.at[pl.ds(last, last_n)],
            out_hbm.at[pl.ds(pl.multiple_of(c * N_EDGES + last, 8), last_n)])


def _no_scale(b, gi, si, rb):
    del b, gi, si, rb


@functools.lru_cache(maxsize=None)
def _build_sc_kernels():
    mesh = plsc.VectorSubcoreMesh(core_axis_name="c", subcore_axis_name="s",
                                  num_cores=NC, num_subcores=NS)

    @functools.partial(
        pl.kernel,
        out_type=jax.ShapeDtypeStruct((NC * N_EDGES, D), jnp.float32),
        mesh=mesh,
        scratch_types=_common_scratch(),
        compiler_params=pltpu.CompilerParams(needs_layout_passes=False),
    )
    def segment_sum_sc(gidx_hbm, sidx_hbm, table_hbm, zeros_hbm, out_hbm,
                       gi0, gi1, si0, si1, rb0, rb1, acc, sem0, sem1):
        _pipeline(table_hbm, gidx_hbm, sidx_hbm, zeros_hbm, out_hbm,
                  (gi0, gi1), (si0, si1), (rb0, rb1), acc, (sem0, sem1),
                  _no_scale)

    @functools.partial(
        pl.kernel,
        out_type=jax.ShapeDtypeStruct((NC * N_NODES, D), jnp.float32),
        mesh=mesh,
        scratch_types=_common_scratch() + [
            pltpu.VMEM((N_EDGES,), jnp.float32),
            pltpu.VMEM((N_NODES,), jnp.float32),
            pltpu.VMEM((K,), jnp.float32),
        ],
        compiler_params=pltpu.CompilerParams(needs_layout_passes=False),
    )
    def att_segment_sum_sc(gidx_hbm, sidx_hbm, table_hbm, zeros_hbm,
                           se_hbm, sn_hbm, out_hbm,
                           gi0, gi1, si0, si1, rb0, rb1, acc, sem0, sem1,
                           se_v, sn_v, att_v):
        pltpu.sync_copy(se_hbm, se_v)
        pltpu.sync_copy(sn_hbm, sn_v)

        def scale(b, gi, si, rb):
            for v in range(K // 16):
                ee = gi[b][pl.ds(v * 16, 16)]
                nn = si[b][pl.ds(v * 16, 16)]
                z = plsc.load_gather(se_v, [ee]) + plsc.load_gather(sn_v, [nn])
                att_v[pl.ds(v * 16, 16)] = jnp.where(z > 0, z, jnp.exp(z) - 1.0)

            def rbody(r, carry):
                a = plsc.load_gather(att_v, [jnp.full((16,), r, jnp.int32)])
                for cb in range(D // 16):
                    rb[b][r, pl.ds(cb * 16, 16)] = (
                        rb[b][r, pl.ds(cb * 16, 16)] * a)
                return carry

            lax.fori_loop(0, K, rbody, 0, unroll=2)

        _pipeline(table_hbm, gidx_hbm, sidx_hbm, zeros_hbm, out_hbm,
                  (gi0, gi1), (si0, si1), (rb0, rb1), acc, (sem0, sem1),
                  scale)

    return segment_sum_sc, att_segment_sum_sc


def _tc1_body(mp_ref, x0_ref, w_ref, aw_ref, xm_ref, se_ref, sn_ref):
    m = mp_ref[0] + mp_ref[1]
    xm = jnp.dot(m, w_ref[...], preferred_element_type=jnp.float32)
    xm_ref[...] = xm
    se_ref[...] = jnp.sum(xm * aw_ref[0][None, :], axis=1, keepdims=True)
    sn_ref[...] = jnp.sum(x0_ref[...] * aw_ref[1][None, :], axis=1,
                          keepdims=True)


_tc1 = pl.pallas_call(
    _tc1_body,
    out_shape=(
        jax.ShapeDtypeStruct((N_EDGES, D), jnp.float32),
        jax.ShapeDtypeStruct((N_EDGES, 1), jnp.float32),
        jax.ShapeDtypeStruct((N_NODES, 1), jnp.float32),
    ),
)


def _tc2_body(p_ref, o_ref):
    o_ref[...] = p_ref[0] + p_ref[1]


_tc2 = pl.pallas_call(
    _tc2_body,
    out_shape=jax.ShapeDtypeStruct((N_NODES, D), jnp.float32),
)


def kernel(x_0, W, att_weight, incidence_indices):
    node_idx = incidence_indices[0]
    edge_idx = incidence_indices[1]
    pad = NNZ_PAD - NNZ
    zero_pad = jnp.zeros((pad,), jnp.int32)
    dummy_pad = jnp.full((pad,), N_EDGES, jnp.int32)
    zeros_rows = jnp.zeros((ZROWS, D), jnp.float32)
    _segment_sum_sc, _att_segment_sum_sc = _build_sc_kernels()

    gidx_a = jnp.concatenate([node_idx, zero_pad])
    sidx_a = jnp.concatenate([edge_idx, dummy_pad])
    m_part = _segment_sum_sc(gidx_a, sidx_a, x_0, zeros_rows)

    xm, se, sn = _tc1(m_part.reshape(NC, N_EDGES, D), x_0, W,
                      att_weight.reshape(2, D))

    gidx_b = jnp.concatenate([edge_idx, zero_pad])
    sidx_b = jnp.concatenate([node_idx, dummy_pad])
    out_part = _att_segment_sum_sc(gidx_b, sidx_b, xm, zeros_rows,
                                   se.reshape(-1), sn.reshape(-1))
    return _tc2(out_part.reshape(NC, N_NODES, D))
